# Initial kernel scaffold; baseline (speedup 1.0000x reference)
import jax
import jax.numpy as jnp
from jax.experimental import pallas as pl


def kernel(x, edge_index, edge_label_index, eps1, W1a, b1a, g1, be1, W1b, b1b, eps2, W2a, b2a, g2, be2, W2b, b2b, gbn1, bbn1, gbn2, bbn2, Wd1, bd1, Wd2, bd2):
    raise NotImplementedError("write your pallas kernel here")


# speedup vs baseline: 3.8315x; 3.8315x over previous
import functools

import jax
import jax.numpy as jnp
import numpy as np
from jax import lax
from jax.experimental import pallas as pl
from jax.experimental.pallas import tpu as pltpu
from jax.experimental.pallas import tpu_sc as plsc

_N = 10000
_E = 320000
_D = 128
_NC = 2
_NS = 16
_NW = _NC * _NS
_K = 80
_EPT = _E // _NW
_CHUNKS = _EPT // _K
_RPT = 624
_RREM = _N - _NS * _RPT

_f32 = jnp.float32


def _mesh():
    return plsc.VectorSubcoreMesh(core_axis_name="c", subcore_axis_name="s")


def _agg_body(x_hbm, src_hbm, dst_hbm, zero_hbm, out_hbm,
              sidx, didx, rows, acc, sem):
    c = lax.axis_index("c")
    s = lax.axis_index("s")
    r0 = s * _RPT
    pltpu.sync_copy(zero_hbm.at[pl.ds(r0, _RPT)], acc.at[pl.ds(r0, _RPT)])

    @pl.when(s == _NS - 1)
    def _():
        pltpu.sync_copy(zero_hbm.at[pl.ds(_NS * _RPT, _RREM)],
                        acc.at[pl.ds(_NS * _RPT, _RREM)])

    plsc.subcore_barrier()

    wid = s * _NC + c
    base0 = wid * _EPT

    def chunk(i, carry):
        b = base0 + i * _K
        pltpu.sync_copy(src_hbm.at[pl.ds(b, _K)], sidx)
        pltpu.async_copy(x_hbm.at[sidx], rows, sem).wait()
        pltpu.sync_copy(dst_hbm.at[pl.ds(b, _K)], didx)
        pltpu.sync_copy(rows, acc.at[didx], add=True)
        return carry

    lax.fori_loop(0, _CHUNKS, chunk, 0)
    plsc.subcore_barrier()
    pltpu.sync_copy(acc.at[pl.ds(r0, _RPT)], out_hbm.at[c, pl.ds(r0, _RPT)])

    @pl.when(s == _NS - 1)
    def _():
        pltpu.sync_copy(acc.at[pl.ds(_NS * _RPT, _RREM)],
                        out_hbm.at[c, pl.ds(_NS * _RPT, _RREM)])


def _sc_scatter_add(x, src, dst, zeros):
    k = pl.kernel(
        _agg_body,
        out_type=jax.ShapeDtypeStruct((_NC, _N, _D), _f32),
        mesh=_mesh(),
        scratch_types=[
            pltpu.VMEM((_K,), jnp.int32),
            pltpu.VMEM((_K,), jnp.int32),
            pltpu.VMEM((_K, _D), _f32),
            pltpu.VMEM_SHARED((_N, _D), _f32),
            pltpu.SemaphoreType.DMA,
        ],
    )
    return k(x, src, dst, zeros)


def _dec_body(a_hbm, b_hbm, l0_hbm, l1_hbm, w2_hbm, bd2_hbm, out_hbm,
              i0, i1, ra, w2v, b2v, ob, sem):
    c = lax.axis_index("c")
    s = lax.axis_index("s")
    pltpu.sync_copy(w2_hbm, w2v)
    pltpu.sync_copy(bd2_hbm, b2v)
    wid = s * _NC + c
    base0 = wid * _EPT
    wblk = [w2v[pl.ds(j * 16, 16)] for j in range(8)]
    b2vec = b2v[...]

    def chunk(i, carry):
        b = base0 + i * _K
        pltpu.sync_copy(l0_hbm.at[pl.ds(b, _K)], i0)
        pltpu.sync_copy(l1_hbm.at[pl.ds(b, _K)], i1)
        pltpu.async_copy(a_hbm.at[i0], ra, sem).wait()
        pltpu.async_copy(b_hbm.at[i1], ra, sem, add=True).wait()

        def grp(g, carry2):
            res = jnp.zeros((16,), _f32)
            for t in range(16):
                e = g * 16 + t
                acc = jnp.zeros((16,), _f32)
                for j in range(8):
                    v = jnp.maximum(ra[e, pl.ds(j * 16, 16)], 0.0)
                    acc = acc + v * wblk[j]
                for sh in (8, 4, 2, 1):
                    perm = lax.iota(jnp.int32, 16) ^ sh
                    acc = acc + acc.at[perm].get(mode="promise_in_bounds")
                res = jnp.where(lax.iota(jnp.int32, 16) == t, acc, res)
            ob[pl.ds(g * 16, 16)] = res + b2vec
            return carry2

        lax.fori_loop(0, _K // 16, grp, 0)
        pltpu.sync_copy(ob, out_hbm.at[pl.ds(b, _K)])
        return carry

    lax.fori_loop(0, _CHUNKS, chunk, 0)


def _sc_decoder(a, b, l0, l1, w2, bd2_16):
    k = pl.kernel(
        _dec_body,
        out_type=jax.ShapeDtypeStruct((_E,), _f32),
        mesh=_mesh(),
        scratch_types=[
            pltpu.VMEM((_K,), jnp.int32),
            pltpu.VMEM((_K,), jnp.int32),
            pltpu.VMEM((_K, _D), _f32),
            pltpu.VMEM((_D,), _f32),
            pltpu.VMEM((16,), _f32),
            pltpu.VMEM((_K,), _f32),
            pltpu.SemaphoreType.DMA,
        ],
    )
    return k(a, b, l0, l1, w2, bd2_16)


def _bn(y, gamma, beta):
    m = jnp.mean(y, axis=0, keepdims=True)
    v = jnp.mean((y - m) * (y - m), axis=0, keepdims=True)
    return (y - m) / jnp.sqrt(v + 1e-5) * gamma + beta


def _dotT(x, w):
    return lax.dot_general(x, w, (((1,), (1,)), ((), ())),
                           preferred_element_type=_f32)


def _gin1_body(x_r, agg_r, eps_r, wa_r, ba_r, g_r, be_r, wb_r, bb_r,
               gbn_r, bbn_r, out_r):
    h = (1.0 + eps_r[0, 0]) * x_r[...] + agg_r[0] + agg_r[1]
    y = _dotT(h, wa_r[...]) + ba_r[...]
    y = jnp.maximum(_bn(y, g_r[...], be_r[...]), 0.0)
    z = _dotT(y, wb_r[...]) + bb_r[...]
    out_r[...] = jnp.maximum(_bn(z, gbn_r[...], bbn_r[...]), 0.0)


def _gin2_body(x_r, agg_r, eps_r, wa_r, ba_r, g_r, be_r, wb_r, bb_r,
               gbn_r, bbn_r, wl_r, wr_r, bd1_r, a_out, b_out):
    h = (1.0 + eps_r[0, 0]) * x_r[...] + agg_r[0] + agg_r[1]
    y = _dotT(h, wa_r[...]) + ba_r[...]
    y = jnp.maximum(_bn(y, g_r[...], be_r[...]), 0.0)
    z = _dotT(y, wb_r[...]) + bb_r[...]
    z = _bn(z, gbn_r[...], bbn_r[...])
    a_out[...] = _dotT(z, wl_r[...]) + bd1_r[...]
    b_out[...] = _dotT(z, wr_r[...])


def _smem_spec():
    return pl.BlockSpec(memory_space=pltpu.SMEM)


def _tc_gin1(x, agg, eps, wa, ba, g, be, wb, bb, gbn, bbn):
    n_in = 11
    specs = [pl.BlockSpec(memory_space=pltpu.VMEM)] * n_in
    specs[2] = _smem_spec()
    return pl.pallas_call(
        _gin1_body,
        out_shape=jax.ShapeDtypeStruct((_N, _D), _f32),
        in_specs=specs,
        out_specs=pl.BlockSpec(memory_space=pltpu.VMEM),
    )(x, agg, eps, wa, ba, g, be, wb, bb, gbn, bbn)


def _tc_gin2(x, agg, eps, wa, ba, g, be, wb, bb, gbn, bbn, wl, wr, bd1):
    n_in = 14
    specs = [pl.BlockSpec(memory_space=pltpu.VMEM)] * n_in
    specs[2] = _smem_spec()
    return pl.pallas_call(
        _gin2_body,
        out_shape=[jax.ShapeDtypeStruct((_N, _D), _f32),
                   jax.ShapeDtypeStruct((_N, _D), _f32)],
        in_specs=specs,
        out_specs=[pl.BlockSpec(memory_space=pltpu.VMEM)] * 2,
    )(x, agg, eps, wa, ba, g, be, wb, bb, gbn, bbn, wl, wr, bd1)


def kernel(x, edge_index, edge_label_index, eps1, W1a, b1a, g1, be1, W1b,
           b1b, eps2, W2a, b2a, g2, be2, W2b, b2b, gbn1, bbn1, gbn2, bbn2,
           Wd1, bd1, Wd2, bd2):
    src = edge_index[0]
    dst = edge_index[1]
    l0 = edge_label_index[0]
    l1 = edge_label_index[1]
    zeros = jnp.zeros((_N, _D), _f32)

    eps1_s = jnp.reshape(eps1, (1, 1))
    eps2_s = jnp.reshape(eps2, (1, 1))

    agg1 = _sc_scatter_add(x, src, dst, zeros)
    h = _tc_gin1(x, agg1, eps1_s, W1a, b1a, g1, be1, W1b, b1b, gbn1, bbn1)
    agg2 = _sc_scatter_add(h, src, dst, zeros)
    wl = Wd1[:, :_D]
    wr = Wd1[:, _D:]
    a, b = _tc_gin2(h, agg2, eps2_s, W2a, b2a, g2, be2, W2b, b2b,
                    gbn2, bbn2, wl, wr, bd1)
    w2 = jnp.reshape(Wd2, (_D,))
    bd2_16 = jnp.broadcast_to(jnp.reshape(bd2, (1,)), (16,))
    out = _sc_decoder(a, b, l0, l1, w2, bd2_16)
    return out


# speedup vs baseline: 5.7461x; 1.4997x over previous
import functools

import jax
import jax.numpy as jnp
import numpy as np
from jax import lax
from jax.experimental import pallas as pl
from jax.experimental.pallas import tpu as pltpu
from jax.experimental.pallas import tpu_sc as plsc

_N = 10000
_E = 320000
_D = 128
_NC = 2
_NS = 16
_NW = _NC * _NS
_K = 80
_EPT = _E // _NW
_CHUNKS = _EPT // _K
_RPT = 624
_RREM = _N - _NS * _RPT

_f32 = jnp.float32


def _mesh():
    return plsc.VectorSubcoreMesh(core_axis_name="c", subcore_axis_name="s")


def _agg_body(x_hbm, src_hbm, dst_hbm, zero_hbm, out_hbm,
              sall, didx0, didx1, rows0, rows1, acc, sem0, sem1, dsem0,
              dsem1):
    c = lax.axis_index("c")
    s = lax.axis_index("s")
    r0 = s * _RPT
    pltpu.sync_copy(zero_hbm.at[pl.ds(r0, _RPT)], acc.at[pl.ds(r0, _RPT)])

    @pl.when(s == _NS - 1)
    def _():
        pltpu.sync_copy(zero_hbm.at[pl.ds(_NS * _RPT, _RREM)],
                        acc.at[pl.ds(_NS * _RPT, _RREM)])

    wid = s * _NC + c
    base0 = wid * _EPT
    pltpu.sync_copy(src_hbm.at[pl.ds(base0, _EPT)], sall)
    plsc.subcore_barrier()

    rows = (rows0, rows1)
    sems = (sem0, sem1)
    didx = (didx0, didx1)
    dsems = (dsem0, dsem1)

    def start(ch, slot):
        pltpu.async_copy(x_hbm.at[sall.at[pl.ds(ch * _K, _K)]], rows[slot],
                         sems[slot])
        pltpu.async_copy(dst_hbm.at[pl.ds(base0 + ch * _K, _K)], didx[slot],
                         dsems[slot])

    def finish(ch, slot):
        pltpu.make_async_copy(x_hbm.at[sall.at[pl.ds(ch * _K, _K)]],
                              rows[slot], sems[slot]).wait()
        pltpu.make_async_copy(dst_hbm.at[pl.ds(base0 + ch * _K, _K)],
                              didx[slot], dsems[slot]).wait()
        pltpu.sync_copy(rows[slot], acc.at[didx[slot]], add=True)

    start(0, 0)

    def body(p, carry):
        c0 = 2 * p
        start(c0 + 1, 1)
        finish(c0, 0)
        start(c0 + 2, 0)
        finish(c0 + 1, 1)
        return carry

    lax.fori_loop(0, (_CHUNKS - 1) // 2, body, 0)
    finish(_CHUNKS - 1, 0)
    plsc.subcore_barrier()
    pltpu.sync_copy(acc.at[pl.ds(r0, _RPT)], out_hbm.at[c, pl.ds(r0, _RPT)])

    @pl.when(s == _NS - 1)
    def _():
        pltpu.sync_copy(acc.at[pl.ds(_NS * _RPT, _RREM)],
                        out_hbm.at[c, pl.ds(_NS * _RPT, _RREM)])


def _sc_scatter_add(x, src, dst, zeros):
    k = pl.kernel(
        _agg_body,
        out_type=jax.ShapeDtypeStruct((_NC, _N, _D), _f32),
        mesh=_mesh(),
        scratch_types=[
            pltpu.VMEM((_EPT,), jnp.int32),
            pltpu.VMEM((_K,), jnp.int32),
            pltpu.VMEM((_K,), jnp.int32),
            pltpu.VMEM((_K, _D), _f32),
            pltpu.VMEM((_K, _D), _f32),
            pltpu.VMEM_SHARED((_N, _D), _f32),
            pltpu.SemaphoreType.DMA,
            pltpu.SemaphoreType.DMA,
            pltpu.SemaphoreType.DMA,
            pltpu.SemaphoreType.DMA,
        ],
    )
    return k(x, src, dst, zeros)


def _dec_body(a_hbm, b_hbm, l0_hbm, l1_hbm, w2_hbm, bd2_hbm, out_hbm,
              i0, i1, ra, w2v, b2v, ob, sem):
    c = lax.axis_index("c")
    s = lax.axis_index("s")
    pltpu.sync_copy(w2_hbm, w2v)
    pltpu.sync_copy(bd2_hbm, b2v)
    wid = s * _NC + c
    base0 = wid * _EPT
    wblk = [w2v[pl.ds(j * 16, 16)] for j in range(8)]
    b2vec = b2v[...]

    def chunk(i, carry):
        b = base0 + i * _K
        pltpu.sync_copy(l0_hbm.at[pl.ds(b, _K)], i0)
        pltpu.sync_copy(l1_hbm.at[pl.ds(b, _K)], i1)
        pltpu.async_copy(a_hbm.at[i0], ra, sem).wait()
        pltpu.async_copy(b_hbm.at[i1], ra, sem, add=True).wait()

        def grp(g, carry2):
            res = jnp.zeros((16,), _f32)
            for t in range(16):
                e = g * 16 + t
                acc = jnp.zeros((16,), _f32)
                for j in range(8):
                    v = jnp.maximum(ra[e, pl.ds(j * 16, 16)], 0.0)
                    acc = acc + v * wblk[j]
                for sh in (8, 4, 2, 1):
                    perm = lax.iota(jnp.int32, 16) ^ sh
                    acc = acc + acc.at[perm].get(mode="promise_in_bounds")
                res = jnp.where(lax.iota(jnp.int32, 16) == t, acc, res)
            ob[pl.ds(g * 16, 16)] = res + b2vec
            return carry2

        lax.fori_loop(0, _K // 16, grp, 0)
        pltpu.sync_copy(ob, out_hbm.at[pl.ds(b, _K)])
        return carry

    lax.fori_loop(0, _CHUNKS, chunk, 0)


def _sc_decoder(a, b, l0, l1, w2, bd2_16):
    k = pl.kernel(
        _dec_body,
        out_type=jax.ShapeDtypeStruct((_E,), _f32),
        mesh=_mesh(),
        scratch_types=[
            pltpu.VMEM((_K,), jnp.int32),
            pltpu.VMEM((_K,), jnp.int32),
            pltpu.VMEM((_K, _D), _f32),
            pltpu.VMEM((_D,), _f32),
            pltpu.VMEM((16,), _f32),
            pltpu.VMEM((_K,), _f32),
            pltpu.SemaphoreType.DMA,
        ],
    )
    return k(a, b, l0, l1, w2, bd2_16)


def _bn(y, gamma, beta):
    m = jnp.mean(y, axis=0, keepdims=True)
    v = jnp.mean((y - m) * (y - m), axis=0, keepdims=True)
    return (y - m) / jnp.sqrt(v + 1e-5) * gamma + beta


def _dotT(x, w):
    return lax.dot_general(x, w, (((1,), (1,)), ((), ())),
                           preferred_element_type=_f32)


def _gin1_body(x_r, agg_r, eps_r, wa_r, ba_r, g_r, be_r, wb_r, bb_r,
               gbn_r, bbn_r, out_r):
    h = (1.0 + eps_r[0, 0]) * x_r[...] + agg_r[0] + agg_r[1]
    y = _dotT(h, wa_r[...]) + ba_r[...]
    y = jnp.maximum(_bn(y, g_r[...], be_r[...]), 0.0)
    z = _dotT(y, wb_r[...]) + bb_r[...]
    out_r[...] = jnp.maximum(_bn(z, gbn_r[...], bbn_r[...]), 0.0)


def _gin2_body(x_r, agg_r, eps_r, wa_r, ba_r, g_r, be_r, wb_r, bb_r,
               gbn_r, bbn_r, wl_r, wr_r, bd1_r, a_out, b_out):
    h = (1.0 + eps_r[0, 0]) * x_r[...] + agg_r[0] + agg_r[1]
    y = _dotT(h, wa_r[...]) + ba_r[...]
    y = jnp.maximum(_bn(y, g_r[...], be_r[...]), 0.0)
    z = _dotT(y, wb_r[...]) + bb_r[...]
    z = _bn(z, gbn_r[...], bbn_r[...])
    a_out[...] = _dotT(z, wl_r[...]) + bd1_r[...]
    b_out[...] = _dotT(z, wr_r[...])


def _smem_spec():
    return pl.BlockSpec(memory_space=pltpu.SMEM)


def _tc_gin1(x, agg, eps, wa, ba, g, be, wb, bb, gbn, bbn):
    n_in = 11
    specs = [pl.BlockSpec(memory_space=pltpu.VMEM)] * n_in
    specs[2] = _smem_spec()
    return pl.pallas_call(
        _gin1_body,
        out_shape=jax.ShapeDtypeStruct((_N, _D), _f32),
        in_specs=specs,
        out_specs=pl.BlockSpec(memory_space=pltpu.VMEM),
    )(x, agg, eps, wa, ba, g, be, wb, bb, gbn, bbn)


def _tc_gin2(x, agg, eps, wa, ba, g, be, wb, bb, gbn, bbn, wl, wr, bd1):
    n_in = 14
    specs = [pl.BlockSpec(memory_space=pltpu.VMEM)] * n_in
    specs[2] = _smem_spec()
    return pl.pallas_call(
        _gin2_body,
        out_shape=[jax.ShapeDtypeStruct((_N, _D), _f32),
                   jax.ShapeDtypeStruct((_N, _D), _f32)],
        in_specs=specs,
        out_specs=[pl.BlockSpec(memory_space=pltpu.VMEM)] * 2,
    )(x, agg, eps, wa, ba, g, be, wb, bb, gbn, bbn, wl, wr, bd1)


def kernel(x, edge_index, edge_label_index, eps1, W1a, b1a, g1, be1, W1b,
           b1b, eps2, W2a, b2a, g2, be2, W2b, b2b, gbn1, bbn1, gbn2, bbn2,
           Wd1, bd1, Wd2, bd2):
    src = edge_index[0]
    dst = edge_index[1]
    l0 = edge_label_index[0]
    l1 = edge_label_index[1]
    zeros = jnp.zeros((_N, _D), _f32)

    eps1_s = jnp.reshape(eps1, (1, 1))
    eps2_s = jnp.reshape(eps2, (1, 1))

    agg1 = _sc_scatter_add(x, src, dst, zeros)
    h = _tc_gin1(x, agg1, eps1_s, W1a, b1a, g1, be1, W1b, b1b, gbn1, bbn1)
    agg2 = _sc_scatter_add(h, src, dst, zeros)
    wl = Wd1[:, :_D]
    wr = Wd1[:, _D:]
    a, b = _tc_gin2(h, agg2, eps2_s, W2a, b2a, g2, be2, W2b, b2b,
                    gbn2, bbn2, wl, wr, bd1)
    w2 = jnp.reshape(Wd2, (_D,))
    bd2_16 = jnp.broadcast_to(jnp.reshape(bd2, (1,)), (16,))
    out = _sc_decoder(a, b, l0, l1, w2, bd2_16)
    return out


# speedup vs baseline: 7.7661x; 1.3515x over previous
import functools

import jax
import jax.numpy as jnp
import numpy as np
from jax import lax
from jax.experimental import pallas as pl
from jax.experimental.pallas import tpu as pltpu
from jax.experimental.pallas import tpu_sc as plsc

_N = 10000
_E = 320000
_D = 128
_NC = 2
_NS = 16
_NW = _NC * _NS
_K = 80
_EPT = _E // _NW
_CHUNKS = _EPT // _K
_RPT = 624
_RREM = _N - _NS * _RPT

_f32 = jnp.float32


def _mesh():
    return plsc.VectorSubcoreMesh(core_axis_name="c", subcore_axis_name="s")


def _agg_body(x_hbm, src_hbm, dst_hbm, zero_hbm, out_hbm,
              sall, didx0, didx1, rows0, rows1, acc, sem0, sem1, dsem0,
              dsem1):
    c = lax.axis_index("c")
    s = lax.axis_index("s")
    r0 = s * _RPT
    pltpu.sync_copy(zero_hbm.at[pl.ds(r0, _RPT)], acc.at[pl.ds(r0, _RPT)])

    @pl.when(s == _NS - 1)
    def _():
        pltpu.sync_copy(zero_hbm.at[pl.ds(_NS * _RPT, _RREM)],
                        acc.at[pl.ds(_NS * _RPT, _RREM)])

    wid = s * _NC + c
    base0 = wid * _EPT
    pltpu.sync_copy(src_hbm.at[pl.ds(base0, _EPT)], sall)
    plsc.subcore_barrier()

    rows = (rows0, rows1)
    sems = (sem0, sem1)
    didx = (didx0, didx1)
    dsems = (dsem0, dsem1)

    def start(ch, slot):
        pltpu.async_copy(x_hbm.at[sall.at[pl.ds(ch * _K, _K)]], rows[slot],
                         sems[slot])
        pltpu.async_copy(dst_hbm.at[pl.ds(base0 + ch * _K, _K)], didx[slot],
                         dsems[slot])

    def finish(ch, slot):
        pltpu.make_async_copy(x_hbm.at[sall.at[pl.ds(ch * _K, _K)]],
                              rows[slot], sems[slot]).wait()
        pltpu.make_async_copy(dst_hbm.at[pl.ds(base0 + ch * _K, _K)],
                              didx[slot], dsems[slot]).wait()
        pltpu.sync_copy(rows[slot], acc.at[didx[slot]], add=True)

    start(0, 0)

    def body(p, carry):
        c0 = 2 * p
        start(c0 + 1, 1)
        finish(c0, 0)
        start(c0 + 2, 0)
        finish(c0 + 1, 1)
        return carry

    lax.fori_loop(0, (_CHUNKS - 1) // 2, body, 0)
    finish(_CHUNKS - 1, 0)
    plsc.subcore_barrier()
    pltpu.sync_copy(acc.at[pl.ds(r0, _RPT)], out_hbm.at[c, pl.ds(r0, _RPT)])

    @pl.when(s == _NS - 1)
    def _():
        pltpu.sync_copy(acc.at[pl.ds(_NS * _RPT, _RREM)],
                        out_hbm.at[c, pl.ds(_NS * _RPT, _RREM)])


def _sc_scatter_add(x, src, dst, zeros):
    k = pl.kernel(
        _agg_body,
        out_type=jax.ShapeDtypeStruct((_NC, _N, _D), _f32),
        mesh=_mesh(),
        scratch_types=[
            pltpu.VMEM((_EPT,), jnp.int32),
            pltpu.VMEM((_K,), jnp.int32),
            pltpu.VMEM((_K,), jnp.int32),
            pltpu.VMEM((_K, _D), _f32),
            pltpu.VMEM((_K, _D), _f32),
            pltpu.VMEM_SHARED((_N, _D), _f32),
            pltpu.SemaphoreType.DMA,
            pltpu.SemaphoreType.DMA,
            pltpu.SemaphoreType.DMA,
            pltpu.SemaphoreType.DMA,
        ],
    )
    return k(x, src, dst, zeros)


def _dec_body(a_hbm, b_hbm, l0_hbm, l1_hbm, w2_hbm, bd2_hbm, out_hbm,
              l0a, l1a, ra0, ra1, w2v, b2v, ob, sem0, sem1):
    c = lax.axis_index("c")
    s = lax.axis_index("s")
    pltpu.sync_copy(w2_hbm, w2v)
    pltpu.sync_copy(bd2_hbm, b2v)
    wid = s * _NC + c
    base0 = wid * _EPT
    pltpu.sync_copy(l0_hbm.at[pl.ds(base0, _EPT)], l0a)
    pltpu.sync_copy(l1_hbm.at[pl.ds(base0, _EPT)], l1a)
    wblk = [w2v[pl.ds(j * 16, 16)] for j in range(8)]
    b2vec = b2v[...]
    ras = (ra0, ra1)
    sems = (sem0, sem1)

    def start_a(ch, slot):
        pltpu.async_copy(a_hbm.at[l0a.at[pl.ds(ch * _K, _K)]], ras[slot],
                         sems[slot])

    def wait_a(ch, slot):
        pltpu.make_async_copy(a_hbm.at[l0a.at[pl.ds(ch * _K, _K)]],
                              ras[slot], sems[slot]).wait()

    def start_b(ch, slot):
        pltpu.async_copy(b_hbm.at[l1a.at[pl.ds(ch * _K, _K)]], ras[slot],
                         sems[slot], add=True)

    def wait_b(ch, slot):
        pltpu.make_async_copy(b_hbm.at[l1a.at[pl.ds(ch * _K, _K)]],
                              ras[slot], sems[slot]).wait()

    def compute(ch, slot):
        ra = ras[slot]

        def grp(g, carry2):
            res = jnp.zeros((16,), _f32)
            for t in range(16):
                e = g * 16 + t
                acc = jnp.zeros((16,), _f32)
                for j in range(8):
                    v = jnp.maximum(ra[e, pl.ds(j * 16, 16)], 0.0)
                    acc = acc + v * wblk[j]
                for sh in (8, 4, 2, 1):
                    perm = lax.iota(jnp.int32, 16) ^ sh
                    acc = acc + acc.at[perm].get(mode="promise_in_bounds")
                res = jnp.where(lax.iota(jnp.int32, 16) == t, acc, res)
            ob[pl.ds(g * 16, 16)] = res + b2vec
            return carry2

        lax.fori_loop(0, _K // 16, grp, 0)
        pltpu.sync_copy(ob, out_hbm.at[pl.ds(base0 + ch * _K, _K)])

    start_a(0, 0)
    wait_a(0, 0)
    start_b(0, 0)
    start_a(1, 1)

    def body(i, carry):
        c0 = 2 * i
        wait_b(c0, 0)
        compute(c0, 0)
        wait_a(c0 + 1, 1)
        start_b(c0 + 1, 1)
        start_a(c0 + 2, 0)
        wait_b(c0 + 1, 1)
        compute(c0 + 1, 1)
        wait_a(c0 + 2, 0)
        start_b(c0 + 2, 0)
        start_a(c0 + 3, 1)
        return carry

    lax.fori_loop(0, (_CHUNKS - 3) // 2, body, 0)
    cl = _CHUNKS - 3
    wait_b(cl, 0)
    compute(cl, 0)
    wait_a(cl + 1, 1)
    start_b(cl + 1, 1)
    start_a(cl + 2, 0)
    wait_b(cl + 1, 1)
    compute(cl + 1, 1)
    wait_a(cl + 2, 0)
    start_b(cl + 2, 0)
    wait_b(cl + 2, 0)
    compute(cl + 2, 0)


def _sc_decoder(a, b, l0, l1, w2, bd2_16):
    k = pl.kernel(
        _dec_body,
        out_type=jax.ShapeDtypeStruct((_E,), _f32),
        mesh=_mesh(),
        scratch_types=[
            pltpu.VMEM((_EPT,), jnp.int32),
            pltpu.VMEM((_EPT,), jnp.int32),
            pltpu.VMEM((_K, _D), _f32),
            pltpu.VMEM((_K, _D), _f32),
            pltpu.VMEM((_D,), _f32),
            pltpu.VMEM((16,), _f32),
            pltpu.VMEM((_K,), _f32),
            pltpu.SemaphoreType.DMA,
            pltpu.SemaphoreType.DMA,
        ],
    )
    return k(a, b, l0, l1, w2, bd2_16)


def _bn(y, gamma, beta):
    m = jnp.mean(y, axis=0, keepdims=True)
    v = jnp.mean((y - m) * (y - m), axis=0, keepdims=True)
    return (y - m) / jnp.sqrt(v + 1e-5) * gamma + beta


def _dotT(x, w):
    return lax.dot_general(x, w, (((1,), (1,)), ((), ())),
                           preferred_element_type=_f32)


def _gin1_body(x_r, agg_r, eps_r, wa_r, ba_r, g_r, be_r, wb_r, bb_r,
               gbn_r, bbn_r, out_r):
    h = (1.0 + eps_r[0, 0]) * x_r[...] + agg_r[0] + agg_r[1]
    y = _dotT(h, wa_r[...]) + ba_r[...]
    y = jnp.maximum(_bn(y, g_r[...], be_r[...]), 0.0)
    z = _dotT(y, wb_r[...]) + bb_r[...]
    out_r[...] = jnp.maximum(_bn(z, gbn_r[...], bbn_r[...]), 0.0)


def _gin2_body(x_r, agg_r, eps_r, wa_r, ba_r, g_r, be_r, wb_r, bb_r,
               gbn_r, bbn_r, wl_r, wr_r, bd1_r, a_out, b_out):
    h = (1.0 + eps_r[0, 0]) * x_r[...] + agg_r[0] + agg_r[1]
    y = _dotT(h, wa_r[...]) + ba_r[...]
    y = jnp.maximum(_bn(y, g_r[...], be_r[...]), 0.0)
    z = _dotT(y, wb_r[...]) + bb_r[...]
    z = _bn(z, gbn_r[...], bbn_r[...])
    a_out[...] = _dotT(z, wl_r[...]) + bd1_r[...]
    b_out[...] = _dotT(z, wr_r[...])


def _smem_spec():
    return pl.BlockSpec(memory_space=pltpu.SMEM)


def _tc_gin1(x, agg, eps, wa, ba, g, be, wb, bb, gbn, bbn):
    n_in = 11
    specs = [pl.BlockSpec(memory_space=pltpu.VMEM)] * n_in
    specs[2] = _smem_spec()
    return pl.pallas_call(
        _gin1_body,
        out_shape=jax.ShapeDtypeStruct((_N, _D), _f32),
        in_specs=specs,
        out_specs=pl.BlockSpec(memory_space=pltpu.VMEM),
    )(x, agg, eps, wa, ba, g, be, wb, bb, gbn, bbn)


def _tc_gin2(x, agg, eps, wa, ba, g, be, wb, bb, gbn, bbn, wl, wr, bd1):
    n_in = 14
    specs = [pl.BlockSpec(memory_space=pltpu.VMEM)] * n_in
    specs[2] = _smem_spec()
    return pl.pallas_call(
        _gin2_body,
        out_shape=[jax.ShapeDtypeStruct((_N, _D), _f32),
                   jax.ShapeDtypeStruct((_N, _D), _f32)],
        in_specs=specs,
        out_specs=[pl.BlockSpec(memory_space=pltpu.VMEM)] * 2,
    )(x, agg, eps, wa, ba, g, be, wb, bb, gbn, bbn, wl, wr, bd1)


def kernel(x, edge_index, edge_label_index, eps1, W1a, b1a, g1, be1, W1b,
           b1b, eps2, W2a, b2a, g2, be2, W2b, b2b, gbn1, bbn1, gbn2, bbn2,
           Wd1, bd1, Wd2, bd2):
    src = edge_index[0]
    dst = edge_index[1]
    l0 = edge_label_index[0]
    l1 = edge_label_index[1]
    zeros = jnp.zeros((_N, _D), _f32)

    eps1_s = jnp.reshape(eps1, (1, 1))
    eps2_s = jnp.reshape(eps2, (1, 1))

    agg1 = _sc_scatter_add(x, src, dst, zeros)
    h = _tc_gin1(x, agg1, eps1_s, W1a, b1a, g1, be1, W1b, b1b, gbn1, bbn1)
    agg2 = _sc_scatter_add(h, src, dst, zeros)
    wl = Wd1[:, :_D]
    wr = Wd1[:, _D:]
    a, b = _tc_gin2(h, agg2, eps2_s, W2a, b2a, g2, be2, W2b, b2b,
                    gbn2, bbn2, wl, wr, bd1)
    w2 = jnp.reshape(Wd2, (_D,))
    bd2_16 = jnp.broadcast_to(jnp.reshape(bd2, (1,)), (16,))
    out = _sc_decoder(a, b, l0, l1, w2, bd2_16)
    return out


# speedup vs baseline: 8.8890x; 1.1446x over previous
import functools

import jax
import jax.numpy as jnp
import numpy as np
from jax import lax
from jax.experimental import pallas as pl
from jax.experimental.pallas import tpu as pltpu
from jax.experimental.pallas import tpu_sc as plsc

_N = 10000
_E = 320000
_D = 128
_NC = 2
_NS = 16
_NW = _NC * _NS
_K = 80
_EPT = _E // _NW
_CHUNKS = _EPT // _K
_RPT = 624
_RREM = _N - _NS * _RPT

_f32 = jnp.float32


def _mesh():
    return plsc.VectorSubcoreMesh(core_axis_name="c", subcore_axis_name="s")


def _agg_body(x_hbm, src_hbm, dst_hbm, zero_hbm, out_hbm,
              sall, didx0, didx1, rows0, rows1, acc, sem0, sem1, dsem0,
              dsem1):
    c = lax.axis_index("c")
    s = lax.axis_index("s")
    r0 = s * _RPT
    pltpu.sync_copy(zero_hbm.at[pl.ds(r0, _RPT)], acc.at[pl.ds(r0, _RPT)])

    @pl.when(s == _NS - 1)
    def _():
        pltpu.sync_copy(zero_hbm.at[pl.ds(_NS * _RPT, _RREM)],
                        acc.at[pl.ds(_NS * _RPT, _RREM)])

    wid = s * _NC + c
    base0 = wid * _EPT
    pltpu.sync_copy(src_hbm.at[pl.ds(base0, _EPT)], sall)
    plsc.subcore_barrier()

    rows = (rows0, rows1)
    sems = (sem0, sem1)
    didx = (didx0, didx1)
    dsems = (dsem0, dsem1)

    def start(ch, slot):
        pltpu.async_copy(x_hbm.at[sall.at[pl.ds(ch * _K, _K)]], rows[slot],
                         sems[slot])
        pltpu.async_copy(dst_hbm.at[pl.ds(base0 + ch * _K, _K)], didx[slot],
                         dsems[slot])

    def finish(ch, slot):
        pltpu.make_async_copy(x_hbm.at[sall.at[pl.ds(ch * _K, _K)]],
                              rows[slot], sems[slot]).wait()
        pltpu.make_async_copy(dst_hbm.at[pl.ds(base0 + ch * _K, _K)],
                              didx[slot], dsems[slot]).wait()
        pltpu.sync_copy(rows[slot], acc.at[didx[slot]], add=True)

    start(0, 0)

    def body(p, carry):
        c0 = 2 * p
        start(c0 + 1, 1)
        finish(c0, 0)
        start(c0 + 2, 0)
        finish(c0 + 1, 1)
        return carry

    lax.fori_loop(0, (_CHUNKS - 1) // 2, body, 0)
    finish(_CHUNKS - 1, 0)
    plsc.subcore_barrier()
    pltpu.sync_copy(acc.at[pl.ds(r0, _RPT)], out_hbm.at[c, pl.ds(r0, _RPT)])

    @pl.when(s == _NS - 1)
    def _():
        pltpu.sync_copy(acc.at[pl.ds(_NS * _RPT, _RREM)],
                        out_hbm.at[c, pl.ds(_NS * _RPT, _RREM)])


def _sc_scatter_add(x, src, dst, zeros):
    k = pl.kernel(
        _agg_body,
        out_type=jax.ShapeDtypeStruct((_NC, _N, _D), _f32),
        mesh=_mesh(),
        scratch_types=[
            pltpu.VMEM((_EPT,), jnp.int32),
            pltpu.VMEM((_K,), jnp.int32),
            pltpu.VMEM((_K,), jnp.int32),
            pltpu.VMEM((_K, _D), _f32),
            pltpu.VMEM((_K, _D), _f32),
            pltpu.VMEM_SHARED((_N, _D), _f32),
            pltpu.SemaphoreType.DMA,
            pltpu.SemaphoreType.DMA,
            pltpu.SemaphoreType.DMA,
            pltpu.SemaphoreType.DMA,
        ],
    )
    return k(x, src, dst, zeros)


_DNB = 3


def _dec_body(a_hbm, b_hbm, l0_hbm, l1_hbm, w2_hbm, bd2_hbm, out_hbm,
              l0a, l1a, ra0, ra1, ra2, rb0, rb1, rb2, w2v, b2v, ob,
              sa0, sa1, sa2, sb0, sb1, sb2):
    c = lax.axis_index("c")
    s = lax.axis_index("s")
    pltpu.sync_copy(w2_hbm, w2v)
    pltpu.sync_copy(bd2_hbm, b2v)
    wid = s * _NC + c
    base0 = wid * _EPT
    pltpu.sync_copy(l0_hbm.at[pl.ds(base0, _EPT)], l0a)
    pltpu.sync_copy(l1_hbm.at[pl.ds(base0, _EPT)], l1a)
    wblk = [w2v[pl.ds(j * 16, 16)] for j in range(8)]
    b2vec = b2v[...]
    ras = (ra0, ra1, ra2)
    rbs = (rb0, rb1, rb2)
    sas = (sa0, sa1, sa2)
    sbs = (sb0, sb1, sb2)
    lane = lax.iota(jnp.int32, 16)
    masks = {sh: (lane & sh) == 0 for sh in (1, 2, 4, 8)}
    perms = {sh: lane ^ sh for sh in (1, 2, 4, 8)}

    def start(ch, slot):
        pltpu.async_copy(a_hbm.at[l0a.at[pl.ds(ch * _K, _K)]], ras[slot],
                         sas[slot])
        pltpu.async_copy(b_hbm.at[l1a.at[pl.ds(ch * _K, _K)]], rbs[slot],
                         sbs[slot])

    def wait(ch, slot):
        pltpu.make_async_copy(a_hbm.at[l0a.at[pl.ds(ch * _K, _K)]],
                              ras[slot], sas[slot]).wait()
        pltpu.make_async_copy(b_hbm.at[l1a.at[pl.ds(ch * _K, _K)]],
                              rbs[slot], sbs[slot]).wait()

    def compute(ch, slot):
        ra = ras[slot]
        rb = rbs[slot]

        def grp(g, carry2):
            accs = []
            for t in range(16):
                e = g * 16 + t
                acc = jnp.zeros((16,), _f32)
                for j in range(8):
                    blk = pl.ds(j * 16, 16)
                    v = jnp.maximum(ra[e, blk] + rb[e, blk], 0.0)
                    acc = acc + v * wblk[j]
                accs.append(acc)
            for sh in (1, 2, 4, 8):
                nxt = []
                for i in range(len(accs) // 2):
                    av, bv = accs[2 * i], accs[2 * i + 1]
                    u = jnp.where(masks[sh], av, bv)
                    v = jnp.where(masks[sh], bv, av)
                    nxt.append(u + v.at[perms[sh]].get(
                        mode="promise_in_bounds"))
                accs = nxt
            ob[pl.ds(g * 16, 16)] = accs[0] + b2vec
            return carry2

        lax.fori_loop(0, _K // 16, grp, 0)
        pltpu.sync_copy(ob, out_hbm.at[pl.ds(base0 + ch * _K, _K)])

    for b in range(_DNB):
        start(b, b)

    def body(p, carry):
        for b in range(_DNB):
            ch = _DNB * p + b
            wait(ch, b)
            compute(ch, b)
            start(ch + _DNB, b)
        return carry

    nfull = (_CHUNKS - 2 * _DNB + 1) // _DNB
    lax.fori_loop(0, nfull, body, 0)
    done = _DNB * nfull
    for i, ch in enumerate(range(done, _CHUNKS)):
        b = ch % _DNB
        wait(ch, b)
        compute(ch, b)
        if ch + _DNB < _CHUNKS:
            start(ch + _DNB, (ch + _DNB) % _DNB)


def _sc_decoder(a, b, l0, l1, w2, bd2_16):
    k = pl.kernel(
        _dec_body,
        out_type=jax.ShapeDtypeStruct((_E,), _f32),
        mesh=_mesh(),
        scratch_types=[
            pltpu.VMEM((_EPT,), jnp.int32),
            pltpu.VMEM((_EPT,), jnp.int32),
            pltpu.VMEM((_K, _D), _f32),
            pltpu.VMEM((_K, _D), _f32),
            pltpu.VMEM((_K, _D), _f32),
            pltpu.VMEM((_K, _D), _f32),
            pltpu.VMEM((_K, _D), _f32),
            pltpu.VMEM((_K, _D), _f32),
            pltpu.VMEM((_D,), _f32),
            pltpu.VMEM((16,), _f32),
            pltpu.VMEM((_K,), _f32),
            pltpu.SemaphoreType.DMA,
            pltpu.SemaphoreType.DMA,
            pltpu.SemaphoreType.DMA,
            pltpu.SemaphoreType.DMA,
            pltpu.SemaphoreType.DMA,
            pltpu.SemaphoreType.DMA,
        ],
    )
    return k(a, b, l0, l1, w2, bd2_16)


def _bn(y, gamma, beta):
    m = jnp.mean(y, axis=0, keepdims=True)
    v = jnp.mean((y - m) * (y - m), axis=0, keepdims=True)
    return (y - m) / jnp.sqrt(v + 1e-5) * gamma + beta


def _dotT(x, w):
    return lax.dot_general(x, w, (((1,), (1,)), ((), ())),
                           preferred_element_type=_f32)


def _gin1_body(x_r, agg_r, eps_r, wa_r, ba_r, g_r, be_r, wb_r, bb_r,
               gbn_r, bbn_r, out_r):
    h = (1.0 + eps_r[0, 0]) * x_r[...] + agg_r[0] + agg_r[1]
    y = _dotT(h, wa_r[...]) + ba_r[...]
    y = jnp.maximum(_bn(y, g_r[...], be_r[...]), 0.0)
    z = _dotT(y, wb_r[...]) + bb_r[...]
    out_r[...] = jnp.maximum(_bn(z, gbn_r[...], bbn_r[...]), 0.0)


def _gin2_body(x_r, agg_r, eps_r, wa_r, ba_r, g_r, be_r, wb_r, bb_r,
               gbn_r, bbn_r, wl_r, wr_r, bd1_r, a_out, b_out):
    h = (1.0 + eps_r[0, 0]) * x_r[...] + agg_r[0] + agg_r[1]
    y = _dotT(h, wa_r[...]) + ba_r[...]
    y = jnp.maximum(_bn(y, g_r[...], be_r[...]), 0.0)
    z = _dotT(y, wb_r[...]) + bb_r[...]
    z = _bn(z, gbn_r[...], bbn_r[...])
    a_out[...] = _dotT(z, wl_r[...]) + bd1_r[...]
    b_out[...] = _dotT(z, wr_r[...])


def _smem_spec():
    return pl.BlockSpec(memory_space=pltpu.SMEM)


def _tc_gin1(x, agg, eps, wa, ba, g, be, wb, bb, gbn, bbn):
    n_in = 11
    specs = [pl.BlockSpec(memory_space=pltpu.VMEM)] * n_in
    specs[2] = _smem_spec()
    return pl.pallas_call(
        _gin1_body,
        out_shape=jax.ShapeDtypeStruct((_N, _D), _f32),
        in_specs=specs,
        out_specs=pl.BlockSpec(memory_space=pltpu.VMEM),
    )(x, agg, eps, wa, ba, g, be, wb, bb, gbn, bbn)


def _tc_gin2(x, agg, eps, wa, ba, g, be, wb, bb, gbn, bbn, wl, wr, bd1):
    n_in = 14
    specs = [pl.BlockSpec(memory_space=pltpu.VMEM)] * n_in
    specs[2] = _smem_spec()
    return pl.pallas_call(
        _gin2_body,
        out_shape=[jax.ShapeDtypeStruct((_N, _D), _f32),
                   jax.ShapeDtypeStruct((_N, _D), _f32)],
        in_specs=specs,
        out_specs=[pl.BlockSpec(memory_space=pltpu.VMEM)] * 2,
    )(x, agg, eps, wa, ba, g, be, wb, bb, gbn, bbn, wl, wr, bd1)


def kernel(x, edge_index, edge_label_index, eps1, W1a, b1a, g1, be1, W1b,
           b1b, eps2, W2a, b2a, g2, be2, W2b, b2b, gbn1, bbn1, gbn2, bbn2,
           Wd1, bd1, Wd2, bd2):
    src = edge_index[0]
    dst = edge_index[1]
    l0 = edge_label_index[0]
    l1 = edge_label_index[1]
    zeros = jnp.zeros((_N, _D), _f32)

    eps1_s = jnp.reshape(eps1, (1, 1))
    eps2_s = jnp.reshape(eps2, (1, 1))

    agg1 = _sc_scatter_add(x, src, dst, zeros)
    h = _tc_gin1(x, agg1, eps1_s, W1a, b1a, g1, be1, W1b, b1b, gbn1, bbn1)
    agg2 = _sc_scatter_add(h, src, dst, zeros)
    wl = Wd1[:, :_D]
    wr = Wd1[:, _D:]
    a, b = _tc_gin2(h, agg2, eps2_s, W2a, b2a, g2, be2, W2b, b2b,
                    gbn2, bbn2, wl, wr, bd1)
    w2 = jnp.reshape(Wd2, (_D,))
    bd2_16 = jnp.broadcast_to(jnp.reshape(bd2, (1,)), (16,))
    out = _sc_decoder(a, b, l0, l1, w2, bd2_16)
    return out


# speedup vs baseline: 9.7920x; 1.1016x over previous
import functools

import jax
import jax.numpy as jnp
import numpy as np
from jax import lax
from jax.experimental import pallas as pl
from jax.experimental.pallas import tpu as pltpu
from jax.experimental.pallas import tpu_sc as plsc

_N = 10000
_E = 320000
_D = 128
_NC = 2
_NS = 16
_NW = _NC * _NS
_K = 80
_EPT = _E // _NW
_CHUNKS = _EPT // _K
_RPT = 624
_RREM = _N - _NS * _RPT

_f32 = jnp.float32

_ANB = 3


def _mesh():
    return plsc.VectorSubcoreMesh(core_axis_name="c", subcore_axis_name="s")


def _agg_body(x_hbm, src_hbm, dst_hbm, zero_hbm, out_hbm,
              sall, didx0, didx1, didx2, rows0, rows1, rows2, acc,
              sem0, sem1, sem2, dsem0, dsem1, dsem2):
    c = lax.axis_index("c")
    s = lax.axis_index("s")
    r0 = s * _RPT
    pltpu.sync_copy(zero_hbm.at[pl.ds(r0, _RPT)], acc.at[pl.ds(r0, _RPT)])

    @pl.when(s == _NS - 1)
    def _():
        pltpu.sync_copy(zero_hbm.at[pl.ds(_NS * _RPT, _RREM)],
                        acc.at[pl.ds(_NS * _RPT, _RREM)])

    wid = s * _NC + c
    base0 = wid * _EPT
    pltpu.sync_copy(src_hbm.at[pl.ds(base0, _EPT)], sall)
    plsc.subcore_barrier()

    rows = (rows0, rows1, rows2)
    sems = (sem0, sem1, sem2)
    didx = (didx0, didx1, didx2)
    dsems = (dsem0, dsem1, dsem2)

    def start(ch, slot):
        pltpu.async_copy(x_hbm.at[sall.at[pl.ds(ch * _K, _K)]], rows[slot],
                         sems[slot])
        pltpu.async_copy(dst_hbm.at[pl.ds(base0 + ch * _K, _K)], didx[slot],
                         dsems[slot])

    def finish(ch, slot):
        pltpu.make_async_copy(x_hbm.at[sall.at[pl.ds(ch * _K, _K)]],
                              rows[slot], sems[slot]).wait()
        pltpu.make_async_copy(dst_hbm.at[pl.ds(base0 + ch * _K, _K)],
                              didx[slot], dsems[slot]).wait()
        pltpu.sync_copy(rows[slot], acc.at[didx[slot]], add=True)

    for b in range(_ANB):
        start(b, b)

    def body(p, carry):
        for b in range(_ANB):
            ch = _ANB * p + b
            finish(ch, b)
            start(ch + _ANB, b)
        return carry

    nfull = (_CHUNKS - 2 * _ANB + 1) // _ANB
    lax.fori_loop(0, nfull, body, 0)
    for ch in range(_ANB * nfull, _CHUNKS):
        finish(ch, ch % _ANB)
        if ch + _ANB < _CHUNKS:
            start(ch + _ANB, (ch + _ANB) % _ANB)
    plsc.subcore_barrier()
    pltpu.sync_copy(acc.at[pl.ds(r0, _RPT)], out_hbm.at[c, pl.ds(r0, _RPT)])

    @pl.when(s == _NS - 1)
    def _():
        pltpu.sync_copy(acc.at[pl.ds(_NS * _RPT, _RREM)],
                        out_hbm.at[c, pl.ds(_NS * _RPT, _RREM)])


def _sc_scatter_add(x, src, dst, zeros):
    k = pl.kernel(
        _agg_body,
        out_type=jax.ShapeDtypeStruct((_NC, _N, _D), _f32),
        mesh=_mesh(),
        scratch_types=[
            pltpu.VMEM((_EPT,), jnp.int32),
            pltpu.VMEM((_K,), jnp.int32),
            pltpu.VMEM((_K,), jnp.int32),
            pltpu.VMEM((_K,), jnp.int32),
            pltpu.VMEM((_K, _D), _f32),
            pltpu.VMEM((_K, _D), _f32),
            pltpu.VMEM((_K, _D), _f32),
            pltpu.VMEM_SHARED((_N, _D), _f32),
            pltpu.SemaphoreType.DMA,
            pltpu.SemaphoreType.DMA,
            pltpu.SemaphoreType.DMA,
            pltpu.SemaphoreType.DMA,
            pltpu.SemaphoreType.DMA,
            pltpu.SemaphoreType.DMA,
        ],
    )
    return k(x, src, dst, zeros)


_DNB = 3


def _dec_body(a_hbm, b_hbm, l0_hbm, l1_hbm, w2_hbm, bd2_hbm, out_hbm,
              l0a, l1a, ra0, ra1, ra2, rb0, rb1, rb2, w2v, b2v, ob,
              sa0, sa1, sa2, sb0, sb1, sb2):
    c = lax.axis_index("c")
    s = lax.axis_index("s")
    pltpu.sync_copy(w2_hbm, w2v)
    pltpu.sync_copy(bd2_hbm, b2v)
    wid = s * _NC + c
    base0 = wid * _EPT
    pltpu.sync_copy(l0_hbm.at[pl.ds(base0, _EPT)], l0a)
    pltpu.sync_copy(l1_hbm.at[pl.ds(base0, _EPT)], l1a)
    wblk = [w2v[pl.ds(j * 16, 16)] for j in range(8)]
    b2vec = b2v[...]
    ras = (ra0, ra1, ra2)
    rbs = (rb0, rb1, rb2)
    sas = (sa0, sa1, sa2)
    sbs = (sb0, sb1, sb2)
    lane = lax.iota(jnp.int32, 16)
    masks = {sh: (lane & sh) == 0 for sh in (1, 2, 4, 8)}
    perms = {sh: lane ^ sh for sh in (1, 2, 4, 8)}

    def start(ch, slot):
        pltpu.async_copy(a_hbm.at[l0a.at[pl.ds(ch * _K, _K)]], ras[slot],
                         sas[slot])
        pltpu.async_copy(b_hbm.at[l1a.at[pl.ds(ch * _K, _K)]], rbs[slot],
                         sbs[slot])

    def wait(ch, slot):
        pltpu.make_async_copy(a_hbm.at[l0a.at[pl.ds(ch * _K, _K)]],
                              ras[slot], sas[slot]).wait()
        pltpu.make_async_copy(b_hbm.at[l1a.at[pl.ds(ch * _K, _K)]],
                              rbs[slot], sbs[slot]).wait()

    def compute(ch, slot):
        ra = ras[slot]
        rb = rbs[slot]

        def grp(g, carry2):
            accs = []
            for t in range(16):
                e = g * 16 + t
                acc = jnp.zeros((16,), _f32)
                for j in range(8):
                    blk = pl.ds(j * 16, 16)
                    v = jnp.maximum(ra[e, blk] + rb[e, blk], 0.0)
                    acc = acc + v * wblk[j]
                accs.append(acc)
            for sh in (1, 2, 4, 8):
                nxt = []
                for i in range(len(accs) // 2):
                    av, bv = accs[2 * i], accs[2 * i + 1]
                    u = jnp.where(masks[sh], av, bv)
                    v = jnp.where(masks[sh], bv, av)
                    nxt.append(u + v.at[perms[sh]].get(
                        mode="promise_in_bounds"))
                accs = nxt
            ob[pl.ds(g * 16, 16)] = accs[0] + b2vec
            return carry2

        lax.fori_loop(0, _K // 16, grp, 0)
        pltpu.sync_copy(ob, out_hbm.at[pl.ds(base0 + ch * _K, _K)])

    for b in range(_DNB):
        start(b, b)

    def body(p, carry):
        for b in range(_DNB):
            ch = _DNB * p + b
            wait(ch, b)
            compute(ch, b)
            start(ch + _DNB, b)
        return carry

    nfull = (_CHUNKS - 2 * _DNB + 1) // _DNB
    lax.fori_loop(0, nfull, body, 0)
    done = _DNB * nfull
    for i, ch in enumerate(range(done, _CHUNKS)):
        b = ch % _DNB
        wait(ch, b)
        compute(ch, b)
        if ch + _DNB < _CHUNKS:
            start(ch + _DNB, (ch + _DNB) % _DNB)


def _sc_decoder(a, b, l0, l1, w2, bd2_16):
    k = pl.kernel(
        _dec_body,
        out_type=jax.ShapeDtypeStruct((_E,), _f32),
        mesh=_mesh(),
        scratch_types=[
            pltpu.VMEM((_EPT,), jnp.int32),
            pltpu.VMEM((_EPT,), jnp.int32),
            pltpu.VMEM((_K, _D), _f32),
            pltpu.VMEM((_K, _D), _f32),
            pltpu.VMEM((_K, _D), _f32),
            pltpu.VMEM((_K, _D), _f32),
            pltpu.VMEM((_K, _D), _f32),
            pltpu.VMEM((_K, _D), _f32),
            pltpu.VMEM((_D,), _f32),
            pltpu.VMEM((16,), _f32),
            pltpu.VMEM((_K,), _f32),
            pltpu.SemaphoreType.DMA,
            pltpu.SemaphoreType.DMA,
            pltpu.SemaphoreType.DMA,
            pltpu.SemaphoreType.DMA,
            pltpu.SemaphoreType.DMA,
            pltpu.SemaphoreType.DMA,
        ],
    )
    return k(a, b, l0, l1, w2, bd2_16)


def _bn(y, gamma, beta):
    m = jnp.mean(y, axis=0, keepdims=True)
    v = jnp.mean((y - m) * (y - m), axis=0, keepdims=True)
    return (y - m) / jnp.sqrt(v + 1e-5) * gamma + beta


def _dotT(x, w):
    return lax.dot_general(x, w, (((1,), (1,)), ((), ())),
                           preferred_element_type=_f32)


def _gin1_body(x_r, agg_r, eps_r, wa_r, ba_r, g_r, be_r, wb_r, bb_r,
               gbn_r, bbn_r, out_r):
    h = (1.0 + eps_r[0, 0]) * x_r[...] + agg_r[0] + agg_r[1]
    y = _dotT(h, wa_r[...]) + ba_r[...]
    y = jnp.maximum(_bn(y, g_r[...], be_r[...]), 0.0)
    z = _dotT(y, wb_r[...]) + bb_r[...]
    out_r[...] = jnp.maximum(_bn(z, gbn_r[...], bbn_r[...]), 0.0)


def _gin2_body(x_r, agg_r, eps_r, wa_r, ba_r, g_r, be_r, wb_r, bb_r,
               gbn_r, bbn_r, wl_r, wr_r, bd1_r, a_out, b_out):
    h = (1.0 + eps_r[0, 0]) * x_r[...] + agg_r[0] + agg_r[1]
    y = _dotT(h, wa_r[...]) + ba_r[...]
    y = jnp.maximum(_bn(y, g_r[...], be_r[...]), 0.0)
    z = _dotT(y, wb_r[...]) + bb_r[...]
    z = _bn(z, gbn_r[...], bbn_r[...])
    a_out[...] = _dotT(z, wl_r[...]) + bd1_r[...]
    b_out[...] = _dotT(z, wr_r[...])


def _smem_spec():
    return pl.BlockSpec(memory_space=pltpu.SMEM)


def _tc_gin1(x, agg, eps, wa, ba, g, be, wb, bb, gbn, bbn):
    n_in = 11
    specs = [pl.BlockSpec(memory_space=pltpu.VMEM)] * n_in
    specs[2] = _smem_spec()
    return pl.pallas_call(
        _gin1_body,
        out_shape=jax.ShapeDtypeStruct((_N, _D), _f32),
        in_specs=specs,
        out_specs=pl.BlockSpec(memory_space=pltpu.VMEM),
    )(x, agg, eps, wa, ba, g, be, wb, bb, gbn, bbn)


def _tc_gin2(x, agg, eps, wa, ba, g, be, wb, bb, gbn, bbn, wl, wr, bd1):
    n_in = 14
    specs = [pl.BlockSpec(memory_space=pltpu.VMEM)] * n_in
    specs[2] = _smem_spec()
    return pl.pallas_call(
        _gin2_body,
        out_shape=[jax.ShapeDtypeStruct((_N, _D), _f32),
                   jax.ShapeDtypeStruct((_N, _D), _f32)],
        in_specs=specs,
        out_specs=[pl.BlockSpec(memory_space=pltpu.VMEM)] * 2,
    )(x, agg, eps, wa, ba, g, be, wb, bb, gbn, bbn, wl, wr, bd1)


def kernel(x, edge_index, edge_label_index, eps1, W1a, b1a, g1, be1, W1b,
           b1b, eps2, W2a, b2a, g2, be2, W2b, b2b, gbn1, bbn1, gbn2, bbn2,
           Wd1, bd1, Wd2, bd2):
    src = edge_index[0]
    dst = edge_index[1]
    l0 = edge_label_index[0]
    l1 = edge_label_index[1]
    zeros = jnp.zeros((_N, _D), _f32)

    eps1_s = jnp.reshape(eps1, (1, 1))
    eps2_s = jnp.reshape(eps2, (1, 1))

    agg1 = _sc_scatter_add(x, src, dst, zeros)
    h = _tc_gin1(x, agg1, eps1_s, W1a, b1a, g1, be1, W1b, b1b, gbn1, bbn1)
    agg2 = _sc_scatter_add(h, src, dst, zeros)
    wl = Wd1[:, :_D]
    wr = Wd1[:, _D:]
    a, b = _tc_gin2(h, agg2, eps2_s, W2a, b2a, g2, be2, W2b, b2b,
                    gbn2, bbn2, wl, wr, bd1)
    w2 = jnp.reshape(Wd2, (_D,))
    bd2_16 = jnp.broadcast_to(jnp.reshape(bd2, (1,)), (16,))
    out = _sc_decoder(a, b, l0, l1, w2, bd2_16)
    return out


# speedup vs baseline: 10.8573x; 1.1088x over previous
import functools

import jax
import jax.numpy as jnp
import numpy as np
from jax import lax
from jax.experimental import pallas as pl
from jax.experimental.pallas import tpu as pltpu
from jax.experimental.pallas import tpu_sc as plsc

_N = 10000
_E = 320000
_D = 128
_NC = 2
_NS = 16
_NW = _NC * _NS
_K = 80
_EPT = _E // _NW
_CHUNKS = _EPT // _K
_RPT = 624
_RREM = _N - _NS * _RPT

_f32 = jnp.float32

_ANB = 3


def _mesh():
    return plsc.VectorSubcoreMesh(core_axis_name="c", subcore_axis_name="s")


def _agg_body(x_hbm, src_hbm, dst_hbm, zero_hbm, out_hbm,
              sall, didx0, didx1, didx2, rows0, rows1, rows2, acc,
              sem0, sem1, sem2, dsem0, dsem1, dsem2):
    c = lax.axis_index("c")
    s = lax.axis_index("s")
    r0 = s * _RPT
    pltpu.sync_copy(zero_hbm.at[pl.ds(r0, _RPT)], acc.at[pl.ds(r0, _RPT)])

    @pl.when(s == _NS - 1)
    def _():
        pltpu.sync_copy(zero_hbm.at[pl.ds(_NS * _RPT, _RREM)],
                        acc.at[pl.ds(_NS * _RPT, _RREM)])

    wid = s * _NC + c
    base0 = wid * _EPT
    pltpu.sync_copy(src_hbm.at[pl.ds(base0, _EPT)], sall)
    plsc.subcore_barrier()

    rows = (rows0, rows1, rows2)
    sems = (sem0, sem1, sem2)
    didx = (didx0, didx1, didx2)
    dsems = (dsem0, dsem1, dsem2)

    def start(ch, slot):
        pltpu.async_copy(x_hbm.at[sall.at[pl.ds(ch * _K, _K)]], rows[slot],
                         sems[slot])
        pltpu.async_copy(dst_hbm.at[pl.ds(base0 + ch * _K, _K)], didx[slot],
                         dsems[slot])

    def finish(ch, slot):
        pltpu.make_async_copy(x_hbm.at[sall.at[pl.ds(ch * _K, _K)]],
                              rows[slot], sems[slot]).wait()
        pltpu.make_async_copy(dst_hbm.at[pl.ds(base0 + ch * _K, _K)],
                              didx[slot], dsems[slot]).wait()
        pltpu.sync_copy(rows[slot], acc.at[didx[slot]], add=True)

    for b in range(_ANB):
        start(b, b)

    def body(p, carry):
        for b in range(_ANB):
            ch = _ANB * p + b
            finish(ch, b)
            start(ch + _ANB, b)
        return carry

    nfull = (_CHUNKS - 2 * _ANB + 1) // _ANB
    lax.fori_loop(0, nfull, body, 0)
    for ch in range(_ANB * nfull, _CHUNKS):
        finish(ch, ch % _ANB)
        if ch + _ANB < _CHUNKS:
            start(ch + _ANB, (ch + _ANB) % _ANB)
    plsc.subcore_barrier()
    pltpu.sync_copy(acc.at[pl.ds(r0, _RPT)], out_hbm.at[c, pl.ds(r0, _RPT)])

    @pl.when(s == _NS - 1)
    def _():
        pltpu.sync_copy(acc.at[pl.ds(_NS * _RPT, _RREM)],
                        out_hbm.at[c, pl.ds(_NS * _RPT, _RREM)])


def _sc_scatter_add(x, src, dst, zeros):
    k = pl.kernel(
        _agg_body,
        out_type=jax.ShapeDtypeStruct((_NC, _N, _D), _f32),
        mesh=_mesh(),
        scratch_types=[
            pltpu.VMEM((_EPT,), jnp.int32),
            pltpu.VMEM((_K,), jnp.int32),
            pltpu.VMEM((_K,), jnp.int32),
            pltpu.VMEM((_K,), jnp.int32),
            pltpu.VMEM((_K, _D), _f32),
            pltpu.VMEM((_K, _D), _f32),
            pltpu.VMEM((_K, _D), _f32),
            pltpu.VMEM_SHARED((_N, _D), _f32),
            pltpu.SemaphoreType.DMA,
            pltpu.SemaphoreType.DMA,
            pltpu.SemaphoreType.DMA,
            pltpu.SemaphoreType.DMA,
            pltpu.SemaphoreType.DMA,
            pltpu.SemaphoreType.DMA,
        ],
    )
    return k(x, src, dst, zeros)


_DNB = 6


def _dec_body(a_hbm, b_hbm, l0_hbm, l1_hbm, w2_hbm, bd2_hbm, out_hbm,
              l0a, l1a, ra0, ra1, ra2, ra3, ra4, ra5, w2v, b2v, ob,
              sa0, sa1, sa2, sa3, sa4, sa5):
    c = lax.axis_index("c")
    s = lax.axis_index("s")
    pltpu.sync_copy(w2_hbm, w2v)
    pltpu.sync_copy(bd2_hbm, b2v)
    wid = s * _NC + c
    base0 = wid * _EPT
    pltpu.sync_copy(l0_hbm.at[pl.ds(base0, _EPT)], l0a)
    pltpu.sync_copy(l1_hbm.at[pl.ds(base0, _EPT)], l1a)
    wblk = [w2v[pl.ds(j * 16, 16)] for j in range(8)]
    b2vec = b2v[...]
    ras = (ra0, ra1, ra2, ra3, ra4, ra5)
    sas = (sa0, sa1, sa2, sa3, sa4, sa5)
    lane = lax.iota(jnp.int32, 16)
    masks = {sh: (lane & sh) == 0 for sh in (1, 2, 4, 8)}
    perms = {sh: lane ^ sh for sh in (1, 2, 4, 8)}

    def start_a(ch, slot):
        pltpu.async_copy(a_hbm.at[l0a.at[pl.ds(ch * _K, _K)]], ras[slot],
                         sas[slot])

    def wait_then_start_b(ch, slot):
        pltpu.make_async_copy(a_hbm.at[l0a.at[pl.ds(ch * _K, _K)]],
                              ras[slot], sas[slot]).wait()
        pltpu.async_copy(b_hbm.at[l1a.at[pl.ds(ch * _K, _K)]], ras[slot],
                         sas[slot], add=True)

    def wait_b(ch, slot):
        pltpu.make_async_copy(b_hbm.at[l1a.at[pl.ds(ch * _K, _K)]],
                              ras[slot], sas[slot]).wait()

    def compute(ch, slot):
        ra = ras[slot]

        def grp(g, carry2):
            accs = []
            for t in range(16):
                e = g * 16 + t
                acc = jnp.zeros((16,), _f32)
                for j in range(8):
                    blk = pl.ds(j * 16, 16)
                    v = jnp.maximum(ra[e, blk], 0.0)
                    acc = acc + v * wblk[j]
                accs.append(acc)
            for sh in (1, 2, 4, 8):
                nxt = []
                for i in range(len(accs) // 2):
                    av, bv = accs[2 * i], accs[2 * i + 1]
                    u = jnp.where(masks[sh], av, bv)
                    v = jnp.where(masks[sh], bv, av)
                    nxt.append(u + v.at[perms[sh]].get(
                        mode="promise_in_bounds"))
                accs = nxt
            ob[pl.ds(g * 16, 16)] = accs[0] + b2vec
            return carry2

        lax.fori_loop(0, _K // 16, grp, 0)
        pltpu.sync_copy(ob, out_hbm.at[pl.ds(base0 + ch * _K, _K)])

    for ch in range(_DNB):
        start_a(ch, ch)
    for ch in range(_DNB // 2):
        wait_then_start_b(ch, ch)

    def body(p, carry):
        for b in range(_DNB):
            ch = _DNB * p + b
            wait_b(ch, b)
            compute(ch, b)
            start_a(ch + _DNB, b)
            wait_then_start_b(ch + _DNB // 2, (b + _DNB // 2) % _DNB)
        return carry

    nfull = (_CHUNKS - 1 - _DNB) // _DNB
    lax.fori_loop(0, nfull, body, 0)
    for ch in range(_DNB * nfull, _CHUNKS):
        wait_b(ch, ch % _DNB)
        compute(ch, ch % _DNB)
        if ch + _DNB < _CHUNKS:
            start_a(ch + _DNB, ch % _DNB)
        if ch + _DNB // 2 < _CHUNKS:
            wait_then_start_b(ch + _DNB // 2, (ch + _DNB // 2) % _DNB)


def _sc_decoder(a, b, l0, l1, w2, bd2_16):
    k = pl.kernel(
        _dec_body,
        out_type=jax.ShapeDtypeStruct((_E,), _f32),
        mesh=_mesh(),
        scratch_types=[
            pltpu.VMEM((_EPT,), jnp.int32),
            pltpu.VMEM((_EPT,), jnp.int32),
            pltpu.VMEM((_K, _D), _f32),
            pltpu.VMEM((_K, _D), _f32),
            pltpu.VMEM((_K, _D), _f32),
            pltpu.VMEM((_K, _D), _f32),
            pltpu.VMEM((_K, _D), _f32),
            pltpu.VMEM((_K, _D), _f32),
            pltpu.VMEM((_D,), _f32),
            pltpu.VMEM((16,), _f32),
            pltpu.VMEM((_K,), _f32),
            pltpu.SemaphoreType.DMA,
            pltpu.SemaphoreType.DMA,
            pltpu.SemaphoreType.DMA,
            pltpu.SemaphoreType.DMA,
            pltpu.SemaphoreType.DMA,
            pltpu.SemaphoreType.DMA,
        ],
    )
    return k(a, b, l0, l1, w2, bd2_16)


def _bn(y, gamma, beta):
    m = jnp.mean(y, axis=0, keepdims=True)
    v = jnp.mean((y - m) * (y - m), axis=0, keepdims=True)
    return (y - m) / jnp.sqrt(v + 1e-5) * gamma + beta


def _dotT(x, w):
    return lax.dot_general(x, w, (((1,), (1,)), ((), ())),
                           preferred_element_type=_f32)


def _gin1_body(x_r, agg_r, eps_r, wa_r, ba_r, g_r, be_r, wb_r, bb_r,
               gbn_r, bbn_r, out_r):
    h = (1.0 + eps_r[0, 0]) * x_r[...] + agg_r[0] + agg_r[1]
    y = _dotT(h, wa_r[...]) + ba_r[...]
    y = jnp.maximum(_bn(y, g_r[...], be_r[...]), 0.0)
    z = _dotT(y, wb_r[...]) + bb_r[...]
    out_r[...] = jnp.maximum(_bn(z, gbn_r[...], bbn_r[...]), 0.0)


def _gin2_body(x_r, agg_r, eps_r, wa_r, ba_r, g_r, be_r, wb_r, bb_r,
               gbn_r, bbn_r, wl_r, wr_r, bd1_r, a_out, b_out):
    h = (1.0 + eps_r[0, 0]) * x_r[...] + agg_r[0] + agg_r[1]
    y = _dotT(h, wa_r[...]) + ba_r[...]
    y = jnp.maximum(_bn(y, g_r[...], be_r[...]), 0.0)
    z = _dotT(y, wb_r[...]) + bb_r[...]
    z = _bn(z, gbn_r[...], bbn_r[...])
    a_out[...] = _dotT(z, wl_r[...]) + bd1_r[...]
    b_out[...] = _dotT(z, wr_r[...])


def _smem_spec():
    return pl.BlockSpec(memory_space=pltpu.SMEM)


def _tc_gin1(x, agg, eps, wa, ba, g, be, wb, bb, gbn, bbn):
    n_in = 11
    specs = [pl.BlockSpec(memory_space=pltpu.VMEM)] * n_in
    specs[2] = _smem_spec()
    return pl.pallas_call(
        _gin1_body,
        out_shape=jax.ShapeDtypeStruct((_N, _D), _f32),
        in_specs=specs,
        out_specs=pl.BlockSpec(memory_space=pltpu.VMEM),
    )(x, agg, eps, wa, ba, g, be, wb, bb, gbn, bbn)


def _tc_gin2(x, agg, eps, wa, ba, g, be, wb, bb, gbn, bbn, wl, wr, bd1):
    n_in = 14
    specs = [pl.BlockSpec(memory_space=pltpu.VMEM)] * n_in
    specs[2] = _smem_spec()
    return pl.pallas_call(
        _gin2_body,
        out_shape=[jax.ShapeDtypeStruct((_N, _D), _f32),
                   jax.ShapeDtypeStruct((_N, _D), _f32)],
        in_specs=specs,
        out_specs=[pl.BlockSpec(memory_space=pltpu.VMEM)] * 2,
    )(x, agg, eps, wa, ba, g, be, wb, bb, gbn, bbn, wl, wr, bd1)


def kernel(x, edge_index, edge_label_index, eps1, W1a, b1a, g1, be1, W1b,
           b1b, eps2, W2a, b2a, g2, be2, W2b, b2b, gbn1, bbn1, gbn2, bbn2,
           Wd1, bd1, Wd2, bd2):
    src = edge_index[0]
    dst = edge_index[1]
    l0 = edge_label_index[0]
    l1 = edge_label_index[1]
    zeros = jnp.zeros((_N, _D), _f32)

    eps1_s = jnp.reshape(eps1, (1, 1))
    eps2_s = jnp.reshape(eps2, (1, 1))

    agg1 = _sc_scatter_add(x, src, dst, zeros)
    h = _tc_gin1(x, agg1, eps1_s, W1a, b1a, g1, be1, W1b, b1b, gbn1, bbn1)
    agg2 = _sc_scatter_add(h, src, dst, zeros)
    wl = Wd1[:, :_D]
    wr = Wd1[:, _D:]
    a, b = _tc_gin2(h, agg2, eps2_s, W2a, b2a, g2, be2, W2b, b2b,
                    gbn2, bbn2, wl, wr, bd1)
    w2 = jnp.reshape(Wd2, (_D,))
    bd2_16 = jnp.broadcast_to(jnp.reshape(bd2, (1,)), (16,))
    out = _sc_decoder(a, b, l0, l1, w2, bd2_16)
    return out
